# baseline (speedup 1.0000x reference)
import functools

import jax
import jax.numpy as jnp
from jax import lax
from jax.experimental import pallas as pl
from jax.experimental.pallas import tpu as pltpu
from jax.experimental.pallas import tpu_sc as plsc

_INFO = plsc.get_sparse_core_info()
_NC, _NS, _L = _INFO.num_cores, _INFO.num_subcores, _INFO.num_lanes
_NW = _NC * _NS

_R = 128
_DP = 385


def _lane_allreduce_sum(v):
    lanes = lax.iota(jnp.int32, _L)
    for k in (8, 4, 2, 1):
        v = v + jnp.take_along_axis(v, lanes ^ k, axis=0)
    return v


def _vrsqrt(sv):
    i = plsc.bitcast(sv, jnp.int32)
    i = jnp.int32(0x5F3759DF) - (i >> 1)
    y = plsc.bitcast(i, jnp.float32)
    half = sv * 0.5
    for _ in range(2):
        y = y * (1.5 - half * y * y)
    return y


_NSTREAM = 4


def _normalize_rows(buf, d):
    ngroups = _R // _L
    zero = jnp.zeros((_L,), jnp.float32)

    @plsc.parallel_loop(0, ngroups, unroll=1)
    def _group(g):
        rows = g * _L + lax.iota(jnp.int32, _L)
        cols0 = tuple(
            jnp.full((_L,), t, jnp.int32) for t in range(_NSTREAM)
        )

        def p1_body(ci, carry):
            cols, accs = carry
            new_cols, new_accs = [], []
            for t in range(_NSTREAM):
                v = plsc.load_gather(buf, [rows, cols[t]])
                new_accs.append(accs[t] + v * v)
                new_cols.append(cols[t] + _NSTREAM)
            return tuple(new_cols), tuple(new_accs)

        _, accs = lax.fori_loop(
            0, d // _NSTREAM, p1_body, (cols0, (zero,) * _NSTREAM),
            unroll=4,
        )
        sv = (accs[0] + accs[1]) + (accs[2] + accs[3])
        sv = jnp.maximum(sv, 1e-24)
        y = _vrsqrt(sv)

        def p2_body(ci, cols):
            new_cols = []
            for t in range(_NSTREAM):
                v = plsc.load_gather(buf, [rows, cols[t]])
                plsc.store_scatter(buf, [rows, cols[t]], v * y)
                new_cols.append(cols[t] + _NSTREAM)
            return tuple(new_cols)

        lax.fori_loop(0, d // _NSTREAM, p2_body, cols0, unroll=4)


def _sc_body(m, d, x_hbm, o_hbm, buf0, buf1, sin0, sin1, sout0, sout1):
    wid = lax.axis_index("s") * _NC + lax.axis_index("c")
    rows_per_w = m // _NW
    base = wid * rows_per_w
    nchunks = rows_per_w // _R
    nvec = d // _L
    bufs = (buf0, buf1)
    sins = (sin0, sin1)
    souts = (sout0, sout1)

    def start_in(k):
        b = k % 2
        return pltpu.async_copy(
            x_hbm.at[pl.ds(base + k * _R, _R)], bufs[b].at[:, pl.ds(0, d)], sins[b]
        )

    def start_out(k):
        b = k % 2
        return pltpu.async_copy(
            bufs[b].at[:, pl.ds(0, d)], o_hbm.at[pl.ds(base + k * _R, _R)], souts[b]
        )

    h_in = [None, None]
    h_out = [None, None]
    h_in[0] = start_in(0)
    for k in range(nchunks):
        b = k % 2
        h_in[b].wait()
        if k + 1 < nchunks:
            if h_out[1 - b] is not None:
                h_out[1 - b].wait()
            h_in[1 - b] = start_in(k + 1)
        _normalize_rows(bufs[b], d)
        h_out[b] = start_out(k)
    h_out[(nchunks - 1) % 2].wait()


def kernel(x):
    m, d = x.shape
    mesh = plsc.VectorSubcoreMesh(core_axis_name="c", subcore_axis_name="s")
    f = pl.kernel(
        functools.partial(_sc_body, m, d),
        out_type=jax.ShapeDtypeStruct((m, d), x.dtype),
        mesh=mesh,
        scratch_types=[
            pltpu.VMEM((_R, _DP), jnp.float32),
            pltpu.VMEM((_R, _DP), jnp.float32),
            pltpu.SemaphoreType.DMA,
            pltpu.SemaphoreType.DMA,
            pltpu.SemaphoreType.DMA,
            pltpu.SemaphoreType.DMA,
        ],
        compiler_params=pltpu.CompilerParams(needs_layout_passes=False),
    )
    return f(x)


# speedup vs baseline: 22.5370x; 22.5370x over previous
import jax
import jax.numpy as jnp
from jax.experimental import pallas as pl


def _norm_body(x_ref, o_ref):
    xb = x_ref[...]
    s = jnp.sum(xb * xb, axis=1, keepdims=True)
    r = jax.lax.rsqrt(jnp.maximum(s, 1e-24))
    o_ref[...] = xb * r


def kernel(x):
    M, D = x.shape
    BM = 8192
    return pl.pallas_call(
        _norm_body,
        grid=(M // BM,),
        in_specs=[pl.BlockSpec((BM, D), lambda i: (i, 0))],
        out_specs=pl.BlockSpec((BM, D), lambda i: (i, 0)),
        out_shape=jax.ShapeDtypeStruct((M, D), x.dtype),
    )(x)
